# baseline (speedup 1.0000x reference)
import jax
import jax.numpy as jnp
from jax.experimental import pallas as pl
from jax.experimental.pallas import tpu as pltpu

N = 10000
D = 128
G = 64
N_CLASS = 10
BM = 200
NS = 2
NI = N // (BM * NS)
F32 = jnp.float32


def _body(adj_a_ref, adj_b_ref, x_ref, idx_ref, w1_ref, b1_ref,
          w2_ref, b2_ref, w3_ref, b3_ref, w4_ref, b4_ref,
          gamma_ref, beta_ref, out_ref, y1_s, z2_s, asum_s, pool_s):
    i = pl.program_id(0)

    @pl.when(i == 0)
    def _init():
        y1_s[...] = (jnp.dot(x_ref[...], w1_ref[...],
                             preferred_element_type=F32) + b1_ref[...])
        asum_s[...] = jnp.zeros_like(asum_s)
        pool_s[...] = jnp.zeros_like(pool_s)

    seg = jax.lax.broadcasted_iota(jnp.int32, (G, BM * NS), 0)
    s_blk = (seg == idx_ref[0]).astype(F32)
    pool_add = jnp.zeros((G, D), F32)
    for h, adj_ref in enumerate((adj_a_ref, adj_b_ref)):
        adj = adj_ref[...]
        s_h = s_blk[:, h * BM:(h + 1) * BM]
        asum_s[...] += jnp.dot(s_h, adj, preferred_element_type=F32)
        p_blk = jnp.dot(adj, y1_s[...], preferred_element_type=F32)
        row0 = i * (BM * NS) + h * BM
        h1 = jnp.maximum(p_blk + y1_s[pl.ds(row0, BM), :], 0.0)
        z2 = (jnp.dot(h1, w2_ref[...], preferred_element_type=F32)
              + b2_ref[...])
        z2_s[pl.ds(row0, BM), :] = z2
        pool_add += jnp.dot(s_h, z2, preferred_element_type=F32)
    pool_s[...] += pool_add

    @pl.when(i == NI - 1)
    def _head():
        pooled = pool_s[...] + jnp.dot(asum_s[...], z2_s[...],
                                       preferred_element_type=F32)
        inv = 1.0 / jnp.sqrt(jnp.float32(1.0 + 1e-5))
        o = gamma_ref[...] * (pooled * inv) + beta_ref[...]
        o = jnp.maximum(jnp.dot(o, w3_ref[...], preferred_element_type=F32)
                        + b3_ref[...], 0.0)
        logits = (jnp.dot(o, w4_ref[...], preferred_element_type=F32)
                  + b4_ref[...])
        m = jnp.max(logits, axis=1, keepdims=True)
        lse = jnp.log(jnp.sum(jnp.exp(logits - m), axis=1, keepdims=True))
        out_ref[...] = logits - m - lse


def kernel(x_in, adj, idx, W1, b1, W2, b2, W3, b3, W4, b4, gamma, beta,
           interpret=False):
    idx3 = idx.reshape(NI, 1, BM * NS)
    full = lambda shape: pl.BlockSpec(shape, lambda i: (0,) * len(shape))

    return pl.pallas_call(
        _body,
        grid=(NI,),
        in_specs=[
            pl.BlockSpec((BM, N), lambda i: (2 * i, 0)),
            pl.BlockSpec((BM, N), lambda i: (2 * i + 1, 0)),
            full((N, D)),
            pl.BlockSpec((1, 1, BM * NS), lambda i: (i, 0, 0)),
            full((D, D)),
            full((1, D)),
            full((D, D)),
            full((1, D)),
            full((D, D)),
            full((1, D)),
            full((D, N_CLASS)),
            full((1, N_CLASS)),
            full((1, D)),
            full((1, D)),
        ],
        out_specs=full((G, N_CLASS)),
        out_shape=jax.ShapeDtypeStruct((G, N_CLASS), F32),
        scratch_shapes=[
            pltpu.VMEM((N, D), F32),
            pltpu.VMEM((N, D), F32),
            pltpu.VMEM((G, N), F32),
            pltpu.VMEM((G, D), F32),
        ],
        compiler_params=pltpu.CompilerParams(
            dimension_semantics=("arbitrary",)),
        interpret=interpret,
    )(adj, adj, x_in, idx3, W1, b1.reshape(1, D), W2, b2.reshape(1, D),
      W3, b3.reshape(1, D), W4, b4.reshape(1, N_CLASS),
      gamma.reshape(1, D), beta.reshape(1, D))


# speedup vs baseline: 1.1719x; 1.1719x over previous
import jax
import jax.numpy as jnp
from jax.experimental import pallas as pl
from jax.experimental.pallas import tpu as pltpu

N = 10000
D = 128
G = 64
N_CLASS = 10
BM = 400
NI = N // BM
F32 = jnp.float32
BF16 = jnp.bfloat16


def _body(adj_ref, x_ref, idx_ref, w1_ref, b1_ref, w2_ref, b2_ref,
          w3_ref, b3_ref, w4_ref, b4_ref, gamma_ref, beta_ref,
          out_ref, y1_s, y1b_s, z2_s, asum_s, pool_s):
    i = pl.program_id(0)

    @pl.when(i == 0)
    def _init():
        y1 = (jnp.dot(x_ref[...], w1_ref[...],
                      preferred_element_type=F32) + b1_ref[...])
        y1_s[...] = y1
        y1b_s[...] = y1.astype(BF16)
        asum_s[...] = jnp.zeros_like(asum_s)
        pool_s[...] = jnp.zeros_like(pool_s)

    adj = adj_ref[...].astype(BF16)
    seg = jax.lax.broadcasted_iota(jnp.int32, (G, BM), 0)
    s_blk = (seg == idx_ref[0]).astype(BF16)
    asum_s[...] += jnp.dot(s_blk, adj, preferred_element_type=F32)

    p_blk = jnp.dot(adj, y1b_s[...], preferred_element_type=F32)
    h1 = jnp.maximum(p_blk + y1_s[pl.ds(i * BM, BM), :], 0.0)
    z2 = (jnp.dot(h1, w2_ref[...], preferred_element_type=F32)
          + b2_ref[...])
    z2_s[pl.ds(i * BM, BM), :] = z2
    pool_s[...] += jnp.dot(s_blk, z2, preferred_element_type=F32)

    @pl.when(i == NI - 1)
    def _head():
        pooled = pool_s[...] + jnp.dot(asum_s[...], z2_s[...],
                                       preferred_element_type=F32)
        inv = 1.0 / jnp.sqrt(jnp.float32(1.0 + 1e-5))
        o = gamma_ref[...] * (pooled * inv) + beta_ref[...]
        o = jnp.maximum(jnp.dot(o, w3_ref[...], preferred_element_type=F32)
                        + b3_ref[...], 0.0)
        logits = (jnp.dot(o, w4_ref[...], preferred_element_type=F32)
                  + b4_ref[...])
        m = jnp.max(logits, axis=1, keepdims=True)
        lse = jnp.log(jnp.sum(jnp.exp(logits - m), axis=1, keepdims=True))
        out_ref[...] = logits - m - lse


def kernel(x_in, adj, idx, W1, b1, W2, b2, W3, b3, W4, b4, gamma, beta,
           interpret=False):
    idx3 = idx.reshape(NI, 1, BM)
    full = lambda shape: pl.BlockSpec(shape, lambda i: (0,) * len(shape))

    return pl.pallas_call(
        _body,
        grid=(NI,),
        in_specs=[
            pl.BlockSpec((BM, N), lambda i: (i, 0)),
            full((N, D)),
            pl.BlockSpec((1, 1, BM), lambda i: (i, 0, 0)),
            full((D, D)),
            full((1, D)),
            full((D, D)),
            full((1, D)),
            full((D, D)),
            full((1, D)),
            full((D, N_CLASS)),
            full((1, N_CLASS)),
            full((1, D)),
            full((1, D)),
        ],
        out_specs=full((G, N_CLASS)),
        out_shape=jax.ShapeDtypeStruct((G, N_CLASS), F32),
        scratch_shapes=[
            pltpu.VMEM((N, D), F32),
            pltpu.VMEM((N, D), BF16),
            pltpu.VMEM((N, D), F32),
            pltpu.VMEM((G, N), F32),
            pltpu.VMEM((G, D), F32),
        ],
        compiler_params=pltpu.CompilerParams(
            dimension_semantics=("arbitrary",)),
        interpret=interpret,
    )(adj, x_in, idx3, W1, b1.reshape(1, D), W2, b2.reshape(1, D),
      W3, b3.reshape(1, D), W4, b4.reshape(1, N_CLASS),
      gamma.reshape(1, D), beta.reshape(1, D))


# speedup vs baseline: 1.1906x; 1.0160x over previous
import jax
import jax.numpy as jnp
from jax.experimental import pallas as pl
from jax.experimental.pallas import tpu as pltpu

N = 10000
D = 128
G = 64
N_CLASS = 10
BM = 400
NI = N // BM
F32 = jnp.float32
BF16 = jnp.bfloat16


def _body(adj_ref, x_ref, idx_ref, w1_ref, b1_ref, w2_ref, b2_ref,
          w3_ref, b3_ref, w4_ref, b4_ref, gamma_ref, beta_ref,
          out_ref, y1_s, z2_s, asum_s, pool_s):
    i = pl.program_id(0)

    @pl.when(i == 0)
    def _init():
        y1_s[...] = (jnp.dot(x_ref[...], w1_ref[...],
                             preferred_element_type=F32) + b1_ref[...])
        asum_s[...] = jnp.zeros_like(asum_s)
        pool_s[...] = jnp.zeros_like(pool_s)

    adj = adj_ref[...]
    seg = jax.lax.broadcasted_iota(jnp.int32, (G, BM), 0)
    s_blk = (seg == idx_ref[0]).astype(F32)
    asum_s[...] += jnp.dot(s_blk, adj, preferred_element_type=F32)

    p_blk = jnp.dot(adj, y1_s[...], preferred_element_type=F32)
    h1 = jnp.maximum(p_blk + y1_s[pl.ds(i * BM, BM), :], 0.0)
    z2 = (jnp.dot(h1, w2_ref[...], preferred_element_type=F32)
          + b2_ref[...])
    z2_s[pl.ds(i * BM, BM), :] = z2.astype(BF16)
    pool_s[...] += jnp.dot(s_blk, z2, preferred_element_type=F32)

    @pl.when(i == NI - 1)
    def _head():
        pooled = pool_s[...] + jnp.dot(asum_s[...].astype(BF16), z2_s[...],
                                       preferred_element_type=F32)
        inv = 1.0 / jnp.sqrt(jnp.float32(1.0 + 1e-5))
        o = gamma_ref[...] * (pooled * inv) + beta_ref[...]
        o = jnp.maximum(jnp.dot(o, w3_ref[...], preferred_element_type=F32)
                        + b3_ref[...], 0.0)
        logits = (jnp.dot(o, w4_ref[...], preferred_element_type=F32)
                  + b4_ref[...])
        m = jnp.max(logits, axis=1, keepdims=True)
        lse = jnp.log(jnp.sum(jnp.exp(logits - m), axis=1, keepdims=True))
        out_ref[...] = logits - m - lse


def kernel(x_in, adj, idx, W1, b1, W2, b2, W3, b3, W4, b4, gamma, beta,
           interpret=False):
    idx3 = idx.reshape(NI, 1, BM)
    full = lambda shape: pl.BlockSpec(shape, lambda i: (0,) * len(shape))

    return pl.pallas_call(
        _body,
        grid=(NI,),
        in_specs=[
            pl.BlockSpec((BM, N), lambda i: (i, 0)),
            full((N, D)),
            pl.BlockSpec((1, 1, BM), lambda i: (i, 0, 0)),
            full((D, D)),
            full((1, D)),
            full((D, D)),
            full((1, D)),
            full((D, D)),
            full((1, D)),
            full((D, N_CLASS)),
            full((1, N_CLASS)),
            full((1, D)),
            full((1, D)),
        ],
        out_specs=full((G, N_CLASS)),
        out_shape=jax.ShapeDtypeStruct((G, N_CLASS), F32),
        scratch_shapes=[
            pltpu.VMEM((N, D), F32),
            pltpu.VMEM((N, D), BF16),
            pltpu.VMEM((G, N), F32),
            pltpu.VMEM((G, D), F32),
        ],
        compiler_params=pltpu.CompilerParams(
            dimension_semantics=("arbitrary",)),
        interpret=interpret,
    )(adj, x_in, idx3, W1, b1.reshape(1, D), W2, b2.reshape(1, D),
      W3, b3.reshape(1, D), W4, b4.reshape(1, N_CLASS),
      gamma.reshape(1, D), beta.reshape(1, D))


# speedup vs baseline: 1.1907x; 1.0000x over previous
import jax
import jax.numpy as jnp
from jax.experimental import pallas as pl
from jax.experimental.pallas import tpu as pltpu

N = 10000
D = 128
G = 64
N_CLASS = 10
BM = 400
NI = N // BM
F32 = jnp.float32
BF16 = jnp.bfloat16


def _body(adj_ref, x_ref, idx_ref, w1_ref, b1_ref, w2_ref, b2_ref,
          w3_ref, b3_ref, w4_ref, b4_ref, gamma_ref, beta_ref,
          out_ref, y1_s, z2_s, asum_s, pool_s):
    i = pl.program_id(0)

    @pl.when(i == 0)
    def _init():
        y1_s[...] = (jnp.dot(x_ref[...], w1_ref[...],
                             preferred_element_type=F32) + b1_ref[...])
        asum_s[...] = jnp.zeros_like(asum_s)
        pool_s[...] = jnp.zeros_like(pool_s)

    adj = adj_ref[...]
    seg = jax.lax.broadcasted_iota(jnp.int32, (G, BM), 0)
    s_blk = (seg == idx_ref[0]).astype(F32)
    asum_s[...] = (asum_s[...].astype(F32)
                   + jnp.dot(s_blk, adj, preferred_element_type=F32)
                   ).astype(BF16)

    p_blk = jnp.dot(adj, y1_s[...], preferred_element_type=F32)
    h1 = jnp.maximum(p_blk + y1_s[pl.ds(i * BM, BM), :], 0.0)
    z2 = (jnp.dot(h1, w2_ref[...], preferred_element_type=F32)
          + b2_ref[...])
    z2_s[pl.ds(i * BM, BM), :] = z2.astype(BF16)
    pool_s[...] += jnp.dot(s_blk, z2, preferred_element_type=F32)

    @pl.when(i == NI - 1)
    def _head():
        pooled = pool_s[...] + jnp.dot(asum_s[...], z2_s[...],
                                       preferred_element_type=F32)
        inv = 1.0 / jnp.sqrt(jnp.float32(1.0 + 1e-5))
        o = gamma_ref[...] * (pooled * inv) + beta_ref[...]
        o = jnp.maximum(jnp.dot(o, w3_ref[...], preferred_element_type=F32)
                        + b3_ref[...], 0.0)
        logits = (jnp.dot(o, w4_ref[...], preferred_element_type=F32)
                  + b4_ref[...])
        m = jnp.max(logits, axis=1, keepdims=True)
        lse = jnp.log(jnp.sum(jnp.exp(logits - m), axis=1, keepdims=True))
        out_ref[...] = logits - m - lse


def kernel(x_in, adj, idx, W1, b1, W2, b2, W3, b3, W4, b4, gamma, beta,
           interpret=False):
    idx3 = idx.reshape(NI, 1, BM)
    full = lambda shape: pl.BlockSpec(shape, lambda i: (0,) * len(shape))

    return pl.pallas_call(
        _body,
        grid=(NI,),
        in_specs=[
            pl.BlockSpec((BM, N), lambda i: (i, 0)),
            full((N, D)),
            pl.BlockSpec((1, 1, BM), lambda i: (i, 0, 0)),
            full((D, D)),
            full((1, D)),
            full((D, D)),
            full((1, D)),
            full((D, D)),
            full((1, D)),
            full((D, N_CLASS)),
            full((1, N_CLASS)),
            full((1, D)),
            full((1, D)),
        ],
        out_specs=full((G, N_CLASS)),
        out_shape=jax.ShapeDtypeStruct((G, N_CLASS), F32),
        scratch_shapes=[
            pltpu.VMEM((N, D), F32),
            pltpu.VMEM((N, D), BF16),
            pltpu.VMEM((G, N), BF16),
            pltpu.VMEM((G, D), F32),
        ],
        compiler_params=pltpu.CompilerParams(
            dimension_semantics=("arbitrary",)),
        interpret=interpret,
    )(adj, x_in, idx3, W1, b1.reshape(1, D), W2, b2.reshape(1, D),
      W3, b3.reshape(1, D), W4, b4.reshape(1, N_CLASS),
      gamma.reshape(1, D), beta.reshape(1, D))


# speedup vs baseline: 1.1924x; 1.0015x over previous
import jax
import jax.numpy as jnp
from jax.experimental import pallas as pl
from jax.experimental.pallas import tpu as pltpu

N = 10000
D = 128
G = 64
N_CLASS = 10
BM = 400
NI = N // BM
F32 = jnp.float32


def _body(adj_ref, x_ref, idx_ref, w1_ref, b1_ref, w2_ref, b2_ref,
          w3_ref, b3_ref, w4_ref, b4_ref, gamma_ref, beta_ref,
          out_ref, y1_s, z2_s, asum_s, pool_s):
    i = pl.program_id(0)

    @pl.when(i == 0)
    def _init():
        y1_s[...] = (jnp.dot(x_ref[...], w1_ref[...],
                             preferred_element_type=F32) + b1_ref[...])
        asum_s[...] = jnp.zeros_like(asum_s)
        pool_s[...] = jnp.zeros_like(pool_s)

    adj = adj_ref[...]
    seg = jax.lax.broadcasted_iota(jnp.int32, (G, BM), 0)
    s_blk = (seg == idx_ref[0]).astype(F32)
    asum_s[...] += jnp.dot(s_blk, adj, preferred_element_type=F32)

    p_blk = jnp.dot(adj, y1_s[...], preferred_element_type=F32)
    h1 = jnp.maximum(p_blk + y1_s[pl.ds(i * BM, BM), :], 0.0)
    z2 = (jnp.dot(h1, w2_ref[...], preferred_element_type=F32)
          + b2_ref[...])
    z2_s[pl.ds(i * BM, BM), :] = z2
    pool_s[...] += jnp.dot(s_blk, z2, preferred_element_type=F32)

    @pl.when(i == NI - 1)
    def _head():
        pooled = pool_s[...] + jnp.dot(asum_s[...], z2_s[...],
                                       preferred_element_type=F32)
        inv = 1.0 / jnp.sqrt(jnp.float32(1.0 + 1e-5))
        o = gamma_ref[...] * (pooled * inv) + beta_ref[...]
        o = jnp.maximum(jnp.dot(o, w3_ref[...], preferred_element_type=F32)
                        + b3_ref[...], 0.0)
        logits = (jnp.dot(o, w4_ref[...], preferred_element_type=F32)
                  + b4_ref[...])
        m = jnp.max(logits, axis=1, keepdims=True)
        lse = jnp.log(jnp.sum(jnp.exp(logits - m), axis=1, keepdims=True))
        out_ref[...] = logits - m - lse


def kernel(x_in, adj, idx, W1, b1, W2, b2, W3, b3, W4, b4, gamma, beta,
           interpret=False):
    idx3 = idx.reshape(NI, 1, BM)
    full = lambda shape: pl.BlockSpec(shape, lambda i: (0,) * len(shape))

    return pl.pallas_call(
        _body,
        grid=(NI,),
        in_specs=[
            pl.BlockSpec((BM, N), lambda i: (i, 0)),
            full((N, D)),
            pl.BlockSpec((1, 1, BM), lambda i: (i, 0, 0)),
            full((D, D)),
            full((1, D)),
            full((D, D)),
            full((1, D)),
            full((D, D)),
            full((1, D)),
            full((D, N_CLASS)),
            full((1, N_CLASS)),
            full((1, D)),
            full((1, D)),
        ],
        out_specs=full((G, N_CLASS)),
        out_shape=jax.ShapeDtypeStruct((G, N_CLASS), F32),
        scratch_shapes=[
            pltpu.VMEM((N, D), F32),
            pltpu.VMEM((N, D), F32),
            pltpu.VMEM((G, N), F32),
            pltpu.VMEM((G, D), F32),
        ],
        compiler_params=pltpu.CompilerParams(
            dimension_semantics=("arbitrary",)),
        interpret=interpret,
    )(adj, x_in, idx3, W1, b1.reshape(1, D), W2, b2.reshape(1, D),
      W3, b3.reshape(1, D), W4, b4.reshape(1, N_CLASS),
      gamma.reshape(1, D), beta.reshape(1, D))


# speedup vs baseline: 1.2328x; 1.0338x over previous
import jax
import jax.numpy as jnp
from jax.experimental import pallas as pl
from jax.experimental.pallas import tpu as pltpu

N = 10000
D = 128
G = 64
N_CLASS = 10
BM = 400
NI = N // BM
F32 = jnp.float32


def _body(adj_ref, x_ref, idx_ref, w1_ref, b1_ref, w2_ref, b2_ref,
          w3_ref, b3_ref, w4_ref, b4_ref, gamma_ref, beta_ref,
          out_ref, y1_s, z2_s, asum_s, pool_s):
    i = pl.program_id(0)

    @pl.when(i == 0)
    def _init():
        y1_s[...] = (jnp.dot(x_ref[...], w1_ref[...],
                             preferred_element_type=F32) + b1_ref[...])
        asum_s[...] = jnp.zeros_like(asum_s)
        pool_s[...] = jnp.zeros_like(pool_s)

    adj = adj_ref[...]
    seg = jax.lax.broadcasted_iota(jnp.int32, (G, BM), 0)
    idx_blk = idx_ref[:, pl.ds(i, 1), :][0]
    s_blk = (seg == idx_blk).astype(F32)
    asum_s[...] += jnp.dot(s_blk, adj, preferred_element_type=F32)

    p_blk = jnp.dot(adj, y1_s[...], preferred_element_type=F32)
    h1 = jnp.maximum(p_blk + y1_s[pl.ds(i * BM, BM), :], 0.0)
    z2 = (jnp.dot(h1, w2_ref[...], preferred_element_type=F32)
          + b2_ref[...])
    z2_s[pl.ds(i * BM, BM), :] = z2
    pool_s[...] += jnp.dot(s_blk, z2, preferred_element_type=F32)

    @pl.when(i == NI - 1)
    def _head():
        pooled = pool_s[...] + jnp.dot(asum_s[...], z2_s[...],
                                       preferred_element_type=F32)
        inv = 1.0 / jnp.sqrt(jnp.float32(1.0 + 1e-5))
        o = gamma_ref[...] * (pooled * inv) + beta_ref[...]
        o = jnp.maximum(jnp.dot(o, w3_ref[...], preferred_element_type=F32)
                        + b3_ref[...], 0.0)
        logits = (jnp.dot(o, w4_ref[...], preferred_element_type=F32)
                  + b4_ref[...])
        m = jnp.max(logits, axis=1, keepdims=True)
        lse = jnp.log(jnp.sum(jnp.exp(logits - m), axis=1, keepdims=True))
        out_ref[...] = logits - m - lse


def kernel(x_in, adj, idx, W1, b1, W2, b2, W3, b3, W4, b4, gamma, beta,
           interpret=False):
    idx3 = idx.reshape(1, NI, BM)
    full = lambda shape: pl.BlockSpec(shape, lambda i: (0,) * len(shape))

    return pl.pallas_call(
        _body,
        grid=(NI,),
        in_specs=[
            pl.BlockSpec((BM, N), lambda i: (i, 0)),
            full((N, D)),
            full((1, NI, BM)),
            full((D, D)),
            full((1, D)),
            full((D, D)),
            full((1, D)),
            full((D, D)),
            full((1, D)),
            full((D, N_CLASS)),
            full((1, N_CLASS)),
            full((1, D)),
            full((1, D)),
        ],
        out_specs=full((G, N_CLASS)),
        out_shape=jax.ShapeDtypeStruct((G, N_CLASS), F32),
        scratch_shapes=[
            pltpu.VMEM((N, D), F32),
            pltpu.VMEM((N, D), F32),
            pltpu.VMEM((G, N), F32),
            pltpu.VMEM((G, D), F32),
        ],
        compiler_params=pltpu.CompilerParams(
            dimension_semantics=("arbitrary",)),
        interpret=interpret,
    )(adj, x_in, idx3, W1, b1.reshape(1, D), W2, b2.reshape(1, D),
      W3, b3.reshape(1, D), W4, b4.reshape(1, N_CLASS),
      gamma.reshape(1, D), beta.reshape(1, D))
